# baseline (speedup 1.0000x reference)
import functools

import jax
import jax.numpy as jnp
from jax import lax
from jax.experimental import pallas as pl
from jax.experimental.pallas import tpu as pltpu
from jax.experimental.pallas import tpu_sc as plsc

NUM_CORES = 2
NUM_SUBCORES = 16
NUM_WORKERS = NUM_CORES * NUM_SUBCORES

NBUF = 4


def _make_emb_kernel(BATCH: int, SEQP: int, SEQ: int, D: int):
  rows_per_w = BATCH // NUM_WORKERS
  assert BATCH % NUM_WORKERS == 0 and rows_per_w % NBUF == 0
  assert rows_per_w // NBUF >= 2
  n_rings = rows_per_w // NBUF
  s0 = min(128, SEQ)
  splits = [(0, s0)]
  if SEQ > 128:
    assert SEQP == 256 and s0 % 8 == 0
    splits.append((s0, SEQ - s0))
  mesh = plsc.VectorSubcoreMesh(core_axis_name="c", subcore_axis_name="s")

  @functools.partial(
      pl.kernel,
      mesh=mesh,
      out_type=jax.ShapeDtypeStruct((BATCH * SEQ, D), jnp.float32),
      compiler_params=pltpu.CompilerParams(use_tc_tiling_on_sc=False),
      scratch_types=[
          pltpu.VMEM((rows_per_w, SEQP), jnp.int32),
          [pltpu.VMEM((SEQ, D), jnp.float32) for _ in range(NBUF)],
          [pltpu.SemaphoreType.DMA for _ in range(NBUF)],
          [pltpu.SemaphoreType.DMA for _ in range(NBUF)],
      ],
  )
  def emb(idx_hbm, table_hbm, out_hbm, idx_v, rows, sem_g, sem_o):
    wid = lax.axis_index("s") * NUM_CORES + lax.axis_index("c")
    base = wid * rows_per_w

    pltpu.sync_copy(idx_hbm.at[pl.ds(base, rows_per_w)], idx_v)

    def gather_descs(b, k):
      return [
          pltpu.make_async_copy(
              table_hbm.at[idx_v.at[b, pl.ds(off, ln)]],
              rows[k].at[pl.ds(off, ln)],
              sem_g[k])
          for off, ln in splits
      ]

    def put_desc(b, k):
      return pltpu.make_async_copy(
          rows[k], out_hbm.at[pl.ds((base + b) * SEQ, SEQ)], sem_o[k])

    def start_gathers(b, k):
      for d in gather_descs(b, k):
        d.start()

    def wait_gathers(b, k):
      for d in gather_descs(b, k):
        d.wait()

    for k in range(NBUF - 1):
      start_gathers(k, k)

    def step(b, k, first, last):
      fb = (k + NBUF - 1) % NBUF
      if not first:
        put_desc(b - 1, fb).wait()
      if not last:
        start_gathers(b + NBUF - 1, fb)
      wait_gathers(b, k)
      put_desc(b, k).start()

    for k in range(NBUF):
      step(k, k, first=(k == 0), last=False)

    def ring(r, carry):
      b0 = r * NBUF
      for k in range(NBUF):
        step(b0 + k, k, first=False, last=False)
      return carry

    lax.fori_loop(1, n_rings - 1, ring, 0)

    b0 = (n_rings - 1) * NBUF
    for k in range(NBUF):
      step(b0 + k, k, first=False, last=(k != 0))

    put_desc(b0 + NBUF - 1, NBUF - 1).wait()

  return emb


@jax.jit
def kernel(input, W):
  D = W.shape[1]
  BATCH, SEQ = input.shape
  idx = input.astype(jnp.int32)
  seqp = SEQ if SEQ <= 128 else 256
  if seqp != SEQ:
    idx = jnp.pad(idx, ((0, 0), (0, seqp - SEQ)))
  emb = _make_emb_kernel(BATCH, seqp, SEQ, D)
  flat = emb(idx, W)
  flat = jax.lax.optimization_barrier(flat)
  return flat.reshape(BATCH, SEQ, D)
